# Initial kernel scaffold; baseline (speedup 1.0000x reference)
import jax
import jax.numpy as jnp
from jax.experimental import pallas as pl


def kernel(x, edge_index, edge_attr, descriptors, batch, lin1_W, lin1_b, g_lin1_W, g_lin2_W, g_att_l, g_att_r, g_bias, gru0_Wih, gru0_bih, gru0_Whh, gru0_bhh, atom_W, atom_att_src, atom_att_dst, atom_b, agru_Wih, agru_bih, agru_Whh, agru_bhh, mol_W, mol_att_src, mol_att_dst, mol_b, mgru_Wih, mgru_bih, mgru_Whh, mgru_bhh, m1_W, m1_b, bn1_g, bn1_b, m2_W, m2_b, bn2_g, bn2_b, m3_W, m3_b, bn3_g, bn3_b, c1_W, c1_b, c2_W, c2_b, c3_W, c3_b):
    raise NotImplementedError("write your pallas kernel here")


# speedup vs baseline: 1.1420x; 1.1420x over previous
import functools
import jax
import jax.numpy as jnp
from jax import lax
from jax.experimental import pallas as pl
from jax.experimental.pallas import tpu as pltpu

N = 10000
E = 320000
H = 128
B = 512


def _leaky(v, s=0.01):
    return jnp.where(v > 0, v, s * v)


def _seg_softmax(a, idx, n):
    m = jax.ops.segment_max(a, idx, num_segments=n)
    m = jnp.where(jnp.isfinite(m), m, 0.0)
    e = jnp.exp(a - m[idx])
    s = jax.ops.segment_sum(e, idx, num_segments=n)
    return e / (s[idx] + 1e-16)


def _gru(inp, hid, Wih, bih, Whh, bhh):
    gi = inp @ Wih.T + bih
    gh = hid @ Whh.T + bhh
    ir, iz, inn = jnp.split(gi, 3, axis=-1)
    hr, hz, hn = jnp.split(gh, 3, axis=-1)
    r = jax.nn.sigmoid(ir + hr)
    z = jax.nn.sigmoid(iz + hz)
    n_ = jnp.tanh(inn + r * hn)
    return (1.0 - z) * n_ + z * hid


def _bn(v, g, b):
    m = v.mean(0)
    var = v.var(0)
    return (v - m) / jnp.sqrt(var + 1e-5) * g + b


def _lin1_body(x_ref, w_ref, b_ref, o_ref):
    y = jnp.dot(x_ref[...], w_ref[...].T, preferred_element_type=jnp.float32)
    y = y + b_ref[...][None, :]
    o_ref[...] = jnp.where(y > 0, y, 0.01 * y)


def _lin1(x, W, b):
    return pl.pallas_call(
        _lin1_body,
        out_shape=jax.ShapeDtypeStruct((N, H), jnp.float32),
    )(x, W, b)


def kernel(x, edge_index, edge_attr, descriptors, batch,
           lin1_W, lin1_b, g_lin1_W, g_lin2_W, g_att_l, g_att_r, g_bias,
           gru0_Wih, gru0_bih, gru0_Whh, gru0_bhh,
           atom_W, atom_att_src, atom_att_dst, atom_b,
           agru_Wih, agru_bih, agru_Whh, agru_bhh,
           mol_W, mol_att_src, mol_att_dst, mol_b,
           mgru_Wih, mgru_bih, mgru_Whh, mgru_bhh,
           m1_W, m1_b, bn1_g, bn1_b,
           m2_W, m2_b, bn2_g, bn2_b,
           m3_W, m3_b, bn3_g, bn3_b,
           c1_W, c1_b, c2_W, c2_b, c3_W, c3_b):
    src = edge_index[0]
    dst = edge_index[1]

    x1 = _lin1(x, lin1_W, lin1_b)

    xe = jnp.concatenate([x1[src], edge_attr], axis=-1)
    xj = _leaky(xe @ g_lin1_W.T)
    alpha = _leaky(xj @ g_att_l + x1[dst] @ g_att_r)
    alpha = _seg_softmax(alpha, dst, N)
    acc = jax.ops.segment_sum(x1[src] * alpha[:, None], dst, num_segments=N)
    h = jax.nn.elu(acc @ g_lin2_W.T + g_bias)
    xc = jax.nn.relu(_gru(h, x1, gru0_Wih, gru0_bih, gru0_Whh, gru0_bhh))

    for l in range(2):
        xp = xc @ atom_W[l].T
        alpha = _leaky((xp @ atom_att_src[l])[src] + (xp @ atom_att_dst[l])[dst])
        alpha = _seg_softmax(alpha, dst, N)
        h = jax.ops.segment_sum(xp[src] * alpha[:, None], dst, num_segments=N) + atom_b[l]
        h = jax.nn.elu(h)
        xc = jax.nn.relu(_gru(h, xc, agru_Wih[l], agru_bih[l], agru_Whh[l], agru_bhh[l]))

    out = jax.nn.relu(jax.ops.segment_sum(xc, batch, num_segments=B))
    xp_s = xc @ mol_W.T
    a_s = xp_s @ mol_att_src
    for _t in range(2):
        a_d = (out @ mol_W.T) @ mol_att_dst
        alpha = _leaky(a_s + a_d[batch])
        alpha = _seg_softmax(alpha, batch, B)
        h = jax.ops.segment_sum(xp_s * alpha[:, None], batch, num_segments=B) + mol_b
        h = jax.nn.elu(h)
        out = jax.nn.relu(_gru(h, out, mgru_Wih, mgru_bih, mgru_Whh, mgru_bhh))

    d = jax.nn.relu(_bn(descriptors @ m1_W.T + m1_b, bn1_g, bn1_b))
    d = jax.nn.relu(_bn(d @ m2_W.T + m2_b, bn2_g, bn2_b))
    d = jax.nn.relu(_bn(d @ m3_W.T + m3_b, bn3_g, bn3_b))
    comb = jnp.concatenate([d, out], axis=1)
    o = jax.nn.relu(comb @ c1_W.T + c1_b)
    o = jax.nn.relu(o @ c2_W.T + c2_b)
    return o @ c3_W.T + c3_b


# speedup vs baseline: 9.6379x; 8.4393x over previous
import functools
import jax
import jax.numpy as jnp
from jax import lax
from jax.experimental import pallas as pl
from jax.experimental.pallas import tpu as pltpu
from jax.experimental.pallas import tpu_sc as plsc

N = 10000
NP = 10240
E = 320000
H = 128
B = 512
ED = 16

NC = 2
NS = 16
EC = E // NC
ET = EC // NS
C = 80
NCHUNK = ET // C
SLICE = NP // NS
G16 = C // 16

_MESH = functools.partial(
    plsc.VectorSubcoreMesh, core_axis_name="c", subcore_axis_name="s")
_SC_PARAMS = pltpu.CompilerParams(needs_layout_passes=False)


def _dot_t(a, w):
    return lax.dot_general(a, w, (((1,), (1,)), ((), ())),
                           preferred_element_type=jnp.float32)


def _leaky(v):
    return jnp.where(v > 0, v, 0.01 * v)


def _elu(v):
    return jnp.where(v > 0, v, jnp.exp(v) - 1.0)


def _sig(v):
    return 1.0 / (1.0 + jnp.exp(-v))


def _gru(inp, hid, Wih, bih, Whh, bhh):
    gi = _dot_t(inp, Wih) + bih
    gh = _dot_t(hid, Whh) + bhh
    r = _sig(gi[:, :H] + gh[:, :H])
    z = _sig(gi[:, H:2 * H] + gh[:, H:2 * H])
    n_ = jnp.tanh(gi[:, 2 * H:] + r * gh[:, 2 * H:])
    return (1.0 - z) * n_ + z * hid


def _bn(v, g, b):
    m = v.mean(0)
    var = v.var(0)
    return (v - m) / jnp.sqrt(var + 1e-5) * g + b


def _prep1_body(x_ref, w_ref, b_ref, wa_ref, ar_ref, x1_ref, u_ref, r_ref):
    x1 = _leaky(_dot_t(x_ref[...], w_ref[...]) + b_ref[...][None, :])
    x1_ref[...] = x1
    u_ref[...] = _dot_t(x1, wa_ref[...])
    r_ref[...] = jnp.dot(x1, ar_ref[...])


def _tc_prep1(x, lin1_W, lin1_b, Wa, att_r):
    return pl.pallas_call(
        _prep1_body,
        out_shape=(
            jax.ShapeDtypeStruct((N, H), jnp.float32),
            jax.ShapeDtypeStruct((N, H), jnp.float32),
            jax.ShapeDtypeStruct((N,), jnp.float32),
        ),
    )(x, lin1_W, lin1_b, Wa, att_r)


_BE = 512


def _ee_gate_body(gu_ref, ea_ref, re_ref, wbt_ref, al_ref, ee_ref):
    v = jnp.dot(ea_ref[...], wbt_ref[...], preferred_element_type=jnp.float32)
    t = _leaky(gu_ref[...] + v)
    ell = _leaky(jnp.dot(t, al_ref[...]) + re_ref[...])
    ee_ref[...] = jnp.exp(ell)


def _tc_ee_gate(gu, ea, re, WbT, att_l):
    grid = E // _BE
    return pl.pallas_call(
        _ee_gate_body,
        grid=(grid,),
        in_specs=[
            pl.BlockSpec((_BE, H), lambda i: (i, 0)),
            pl.BlockSpec((_BE, ED), lambda i: (i, 0)),
            pl.BlockSpec((_BE,), lambda i: (i,)),
            pl.BlockSpec((ED, H), lambda i: (0, 0)),
            pl.BlockSpec((H,), lambda i: (0,)),
        ],
        out_specs=pl.BlockSpec((_BE,), lambda i: (i,)),
        out_shape=jax.ShapeDtypeStruct((E,), jnp.float32),
    )(gu, ea, re, WbT, att_l)


def _post_core(accsum, h, xprev, wih, bih, whh, bhh, nw, nas, nad,
               x_ref, xp_ref, as_ref, ad_ref):
    xnew = jnp.maximum(_gru(h, xprev, wih, bih, whh, bhh), 0.0)
    x_ref[...] = xnew
    xp = _dot_t(xnew, nw)
    xp_ref[...] = xp
    as_ref[...] = jnp.dot(xp, nas)
    ad_ref[...] = jnp.dot(xp, nad)


def _post_gate_body(acc_ref, x_ref, w2_ref, gb_ref, wih_ref, bih_ref,
                    whh_ref, bhh_ref, nw_ref, nas_ref, nad_ref,
                    xo_ref, xp_ref, as_ref, ad_ref):
    accsum = acc_ref[0, :N, :] + acc_ref[1, :N, :]
    h = _elu(_dot_t(accsum, w2_ref[...]) + gb_ref[...][None, :])
    _post_core(accsum, h, x_ref[...], wih_ref[...], bih_ref[...],
               whh_ref[...], bhh_ref[...], nw_ref[...], nas_ref[...],
               nad_ref[...], xo_ref, xp_ref, as_ref, ad_ref)


def _post_atom_body(acc_ref, x_ref, ab_ref, wih_ref, bih_ref,
                    whh_ref, bhh_ref, nw_ref, nas_ref, nad_ref,
                    xo_ref, xp_ref, as_ref, ad_ref):
    accsum = acc_ref[0, :N, :] + acc_ref[1, :N, :]
    h = _elu(accsum + ab_ref[...][None, :])
    _post_core(accsum, h, x_ref[...], wih_ref[...], bih_ref[...],
               whh_ref[...], bhh_ref[...], nw_ref[...], nas_ref[...],
               nad_ref[...], xo_ref, xp_ref, as_ref, ad_ref)


_POST_OUT = (
    jax.ShapeDtypeStruct((N, H), jnp.float32),
    jax.ShapeDtypeStruct((N, H), jnp.float32),
    jax.ShapeDtypeStruct((N,), jnp.float32),
    jax.ShapeDtypeStruct((N,), jnp.float32),
)


def _tc_post_gate(acc, x1, w2, gb, wih, bih, whh, bhh, nw, nas, nad):
    return pl.pallas_call(_post_gate_body, out_shape=_POST_OUT)(
        acc, x1, w2, gb, wih, bih, whh, bhh, nw, nas, nad)


def _tc_post_atom(acc, xprev, ab, wih, bih, whh, bhh, nw, nas, nad):
    return pl.pallas_call(_post_atom_body, out_shape=_POST_OUT)(
        acc, xprev, ab, wih, bih, whh, bhh, nw, nas, nad)


_CW = 1280


def _build_z(batch_ref, z_ref):
    for k in range(NP // _CW):
        sl = pl.ds(k * _CW, _CW)
        bi = batch_ref[sl]
        io2 = lax.broadcasted_iota(jnp.int32, (B, _CW), 0)
        z_ref[:, sl] = (bi[None, :] == io2).astype(jnp.float32)


def _mol0_body(x_ref, batch_ref, o_ref, z_ref):
    _build_z(batch_ref, z_ref)
    o_ref[...] = jnp.maximum(
        jnp.dot(z_ref[...], x_ref[...], preferred_element_type=jnp.float32),
        0.0)


def _tc_mol0(x4p, batch_p):
    return pl.pallas_call(
        _mol0_body,
        out_shape=jax.ShapeDtypeStruct((B, H), jnp.float32),
        scratch_shapes=[pltpu.VMEM((B, NP), jnp.float32)],
    )(x4p, batch_p)


def _mol_body(xps_ref, as_ref, batch_ref, out0_ref, desc_ref,
              molW_ref, mad_ref, mb_ref,
              mwih_ref, mbih_ref, mwhh_ref, mbhh_ref,
              m1w_ref, m1b_ref, g1_ref, b1_ref,
              m2w_ref, m2b_ref, g2_ref, b2_ref,
              m3w_ref, m3b_ref, g3_ref, b3_ref,
              c1w_ref, c1b_ref, c2w_ref, c2b_ref, c3w_ref,
              o_ref, z_ref):
    xps = xps_ref[...]
    a_s = as_ref[...]
    _build_z(batch_ref, z_ref)
    Z = z_ref[...]

    out = out0_ref[...]
    for _t in range(2):
        a_d = jnp.dot(_dot_t(out, molW_ref[...]), mad_ref[...])
        ell = _leaky(a_s + jnp.dot(a_d, Z))
        ee = jnp.exp(ell)
        s_b = jnp.dot(Z, ee)
        alpha = ee / (jnp.dot(s_b, Z) + 1e-16)
        h = jnp.dot(Z, xps * alpha[:, None],
                    preferred_element_type=jnp.float32) + mb_ref[...][None, :]
        h = _elu(h)
        out = jnp.maximum(
            _gru(h, out, mwih_ref[...], mbih_ref[...], mwhh_ref[...],
                 mbhh_ref[...]), 0.0)

    d = jnp.maximum(_bn(_dot_t(desc_ref[...], m1w_ref[...]) + m1b_ref[...],
                        g1_ref[...], b1_ref[...]), 0.0)
    d = jnp.maximum(_bn(_dot_t(d, m2w_ref[...]) + m2b_ref[...],
                        g2_ref[...], b2_ref[...]), 0.0)
    d = jnp.maximum(_bn(_dot_t(d, m3w_ref[...]) + m3b_ref[...],
                        g3_ref[...], b3_ref[...]), 0.0)
    comb = jnp.concatenate([d, out], axis=1)
    o = jnp.maximum(_dot_t(comb, c1w_ref[...]) + c1b_ref[...], 0.0)
    o = jnp.maximum(_dot_t(o, c2w_ref[...]) + c2b_ref[...], 0.0)
    o_ref[...] = jnp.dot(o, c3w_ref[0, :])


def _tc_mol(xps, a_s, batch, out0, desc, molW, mad, mb,
            mwih, mbih, mwhh, mbhh, m1w, m1b, g1, b1, m2w, m2b, g2, b2,
            m3w, m3b, g3, b3, c1w, c1b, c2w, c2b, c3w):
    return pl.pallas_call(
        _mol_body,
        out_shape=jax.ShapeDtypeStruct((B,), jnp.float32),
        scratch_shapes=[pltpu.VMEM((B, NP), jnp.float32)],
    )(xps, a_s, batch, out0, desc, molW, mad, mb, mwih, mbih, mwhh, mbhh,
      m1w, m1b, g1, b1, m2w, m2b, g2, b2, m3w, m3b, g3, b3,
      c1w, c1b, c2w, c2b, c3w)


def _sc_gather_body(u_h, r_h, src_h, dst_h, gu_h, re_h,
                    r_vm, srcb, dstb, reb, rows, sem):
    c = lax.axis_index("c")
    s = lax.axis_index("s")
    tbase = c * EC + s * ET
    pltpu.sync_copy(r_h, r_vm.at[pl.ds(0, N)])

    def chunk(j, carry):
        base = tbase + j * C
        pltpu.sync_copy(src_h.at[pl.ds(base, C)], srcb)
        pltpu.sync_copy(dst_h.at[pl.ds(base, C)], dstb)
        pltpu.async_copy(u_h.at[srcb], rows, sem).wait()

        def grp(g, carry2):
            di = dstb[pl.ds(g * 16, 16)]
            reb[pl.ds(g * 16, 16)] = plsc.load_gather(r_vm, [di])
            return carry2
        lax.fori_loop(0, G16, grp, 0)
        pltpu.sync_copy(rows, gu_h.at[pl.ds(base, C), :])
        pltpu.sync_copy(reb, re_h.at[pl.ds(base, C)])
        return carry
    lax.fori_loop(0, NCHUNK, chunk, 0)


def _sc_gather_gate(u, r, src, dst):
    f = pl.kernel(
        _sc_gather_body,
        out_type=(
            jax.ShapeDtypeStruct((E, H), jnp.float32),
            jax.ShapeDtypeStruct((E,), jnp.float32),
        ),
        mesh=_MESH(),
        compiler_params=_SC_PARAMS,
        scratch_types=[
            pltpu.VMEM((NP,), jnp.float32),
            pltpu.VMEM((C,), jnp.int32),
            pltpu.VMEM((C,), jnp.int32),
            pltpu.VMEM((C,), jnp.float32),
            pltpu.VMEM((C, H), jnp.float32),
            pltpu.SemaphoreType.DMA,
        ],
    )
    return f(u, r, src, dst)


def _zero_vec(ref, nwords):
    z = jnp.zeros((16,), jnp.float32)

    def body(i, carry):
        ref[pl.ds(i * 16, 16)] = z
        return carry
    lax.fori_loop(0, nwords // 16, body, 0)


def _sc_p1_atom_body(as_h, ad_h, src_h, dst_h, ee_h, s_h,
                     av_vm, bv_vm, srcb, dstb, eeb, zb, s_spm):
    c = lax.axis_index("c")
    s = lax.axis_index("s")
    tbase = c * EC + s * ET
    pltpu.sync_copy(as_h, av_vm.at[pl.ds(0, N)])
    pltpu.sync_copy(ad_h, bv_vm.at[pl.ds(0, N)])
    _zero_vec(zb, SLICE)
    pltpu.sync_copy(zb, s_spm.at[pl.ds(s * SLICE, SLICE)])
    plsc.subcore_barrier()

    def chunk(j, carry):
        base = tbase + j * C
        pltpu.sync_copy(src_h.at[pl.ds(base, C)], srcb)
        pltpu.sync_copy(dst_h.at[pl.ds(base, C)], dstb)

        def grp(g, carry2):
            sl = pl.ds(g * 16, 16)
            a = (plsc.load_gather(av_vm, [srcb[sl]])
                 + plsc.load_gather(bv_vm, [dstb[sl]]))
            eeb[sl] = jnp.exp(jnp.where(a > 0, a, 0.01 * a))
            return carry2
        lax.fori_loop(0, G16, grp, 0)
        pltpu.sync_copy(eeb, ee_h.at[pl.ds(base, C)])
        pltpu.sync_copy(eeb, s_spm.at[dstb], add=True)
        return carry
    lax.fori_loop(0, NCHUNK, chunk, 0)
    plsc.subcore_barrier()
    pltpu.sync_copy(s_spm.at[pl.ds(s * SLICE, SLICE)], zb)
    pltpu.sync_copy(zb, s_h.at[c, pl.ds(s * SLICE, SLICE)])


def _sc_p1_atom(asrc, adst, src, dst):
    f = pl.kernel(
        _sc_p1_atom_body,
        out_type=(
            jax.ShapeDtypeStruct((E,), jnp.float32),
            jax.ShapeDtypeStruct((NC, NP), jnp.float32),
        ),
        mesh=_MESH(),
        compiler_params=_SC_PARAMS,
        scratch_types=[
            pltpu.VMEM((NP,), jnp.float32),
            pltpu.VMEM((NP,), jnp.float32),
            pltpu.VMEM((C,), jnp.int32),
            pltpu.VMEM((C,), jnp.int32),
            pltpu.VMEM((C,), jnp.float32),
            pltpu.VMEM((SLICE,), jnp.float32),
            pltpu.VMEM_SHARED((NP,), jnp.float32),
        ],
    )
    return f(asrc, adst, src, dst)


def _sc_seg_body(ee_h, dst_h, s_h, dstb, eeb, zb, s_spm):
    c = lax.axis_index("c")
    s = lax.axis_index("s")
    tbase = c * EC + s * ET
    _zero_vec(zb, SLICE)
    pltpu.sync_copy(zb, s_spm.at[pl.ds(s * SLICE, SLICE)])
    plsc.subcore_barrier()

    def chunk(j, carry):
        base = tbase + j * C
        pltpu.sync_copy(dst_h.at[pl.ds(base, C)], dstb)
        pltpu.sync_copy(ee_h.at[pl.ds(base, C)], eeb)
        pltpu.sync_copy(eeb, s_spm.at[dstb], add=True)
        return carry
    lax.fori_loop(0, NCHUNK, chunk, 0)
    plsc.subcore_barrier()
    pltpu.sync_copy(s_spm.at[pl.ds(s * SLICE, SLICE)], zb)
    pltpu.sync_copy(zb, s_h.at[c, pl.ds(s * SLICE, SLICE)])


def _sc_segsum(ee, dst):
    f = pl.kernel(
        _sc_seg_body,
        out_type=jax.ShapeDtypeStruct((NC, NP), jnp.float32),
        mesh=_MESH(),
        compiler_params=_SC_PARAMS,
        scratch_types=[
            pltpu.VMEM((C,), jnp.int32),
            pltpu.VMEM((C,), jnp.float32),
            pltpu.VMEM((SLICE,), jnp.float32),
            pltpu.VMEM_SHARED((NP,), jnp.float32),
        ],
    )
    return f(ee, dst)


def _sc_rows_body(tab_h, src_h, dst_h, ee_h, s_h, acc_h,
                  s_vm, s2_vm, srcb, dstb, eeb, alb, rows, zrow, acc_spm,
                  sem):
    c = lax.axis_index("c")
    s = lax.axis_index("s")
    tbase = c * EC + s * ET
    pltpu.sync_copy(s_h.at[0], s_vm)
    pltpu.sync_copy(s_h.at[1], s2_vm)

    def addv(i, carry):
        sl = pl.ds(i * 16, 16)
        s_vm[sl] = s_vm[sl] + s2_vm[sl]
        return carry
    lax.fori_loop(0, NP // 16, addv, 0)

    z = jnp.zeros((16,), jnp.float32)

    def zrow_i(i, carry):
        for k in range(H // 16):
            zrow[i, pl.ds(k * 16, 16)] = z
        return carry
    lax.fori_loop(0, C, zrow_i, 0)

    def zcp(b_, carry):
        pltpu.sync_copy(zrow, acc_spm.at[pl.ds(s * SLICE + b_ * C, C), :])
        return carry
    lax.fori_loop(0, SLICE // C, zcp, 0)
    plsc.subcore_barrier()

    def chunk(j, carry):
        base = tbase + j * C
        pltpu.sync_copy(src_h.at[pl.ds(base, C)], srcb)
        pltpu.sync_copy(dst_h.at[pl.ds(base, C)], dstb)
        pltpu.sync_copy(ee_h.at[pl.ds(base, C)], eeb)

        def grp(g, carry2):
            sl = pl.ds(g * 16, 16)
            sv = plsc.load_gather(s_vm, [dstb[sl]])
            alb[sl] = eeb[sl] / (sv + 1e-16)
            return carry2
        lax.fori_loop(0, G16, grp, 0)
        pltpu.async_copy(tab_h.at[srcb], rows, sem).wait()

        def scale(i, carry2):
            a = alb[pl.ds(i, 16)][0]
            for k in range(H // 16):
                sl = pl.ds(k * 16, 16)
                rows[i, sl] = rows[i, sl] * a
            return carry2
        lax.fori_loop(0, C, scale, 0)
        pltpu.sync_copy(rows, acc_spm.at[dstb], add=True)
        return carry
    lax.fori_loop(0, NCHUNK, chunk, 0)
    plsc.subcore_barrier()

    def out_cp(b_, carry):
        off = s * SLICE + b_ * C
        pltpu.sync_copy(acc_spm.at[pl.ds(off, C), :], rows)
        pltpu.sync_copy(rows, acc_h.at[c, pl.ds(off, C), :])
        return carry
    lax.fori_loop(0, SLICE // C, out_cp, 0)


def _sc_rows(table, src, dst, ee, s2):
    f = pl.kernel(
        _sc_rows_body,
        out_type=jax.ShapeDtypeStruct((NC, NP, H), jnp.float32),
        mesh=_MESH(),
        compiler_params=_SC_PARAMS,
        scratch_types=[
            pltpu.VMEM((NP,), jnp.float32),
            pltpu.VMEM((NP,), jnp.float32),
            pltpu.VMEM((C,), jnp.int32),
            pltpu.VMEM((C,), jnp.int32),
            pltpu.VMEM((C,), jnp.float32),
            pltpu.VMEM((C + 16,), jnp.float32),
            pltpu.VMEM((C, H), jnp.float32),
            pltpu.VMEM((C, H), jnp.float32),
            pltpu.VMEM_SHARED((NP, H), jnp.float32),
            pltpu.SemaphoreType.DMA,
        ],
    )
    return f(table, src, dst, ee, s2)


def kernel(x, edge_index, edge_attr, descriptors, batch,
           lin1_W, lin1_b, g_lin1_W, g_lin2_W, g_att_l, g_att_r, g_bias,
           gru0_Wih, gru0_bih, gru0_Whh, gru0_bhh,
           atom_W, atom_att_src, atom_att_dst, atom_b,
           agru_Wih, agru_bih, agru_Whh, agru_bhh,
           mol_W, mol_att_src, mol_att_dst, mol_b,
           mgru_Wih, mgru_bih, mgru_Whh, mgru_bhh,
           m1_W, m1_b, bn1_g, bn1_b,
           m2_W, m2_b, bn2_g, bn2_b,
           m3_W, m3_b, bn3_g, bn3_b,
           c1_W, c1_b, c2_W, c2_b, c3_W, c3_b):
    src = edge_index[0]
    dst = edge_index[1]
    Wa = g_lin1_W[:, :H]
    WbT = g_lin1_W[:, H:].T

    x1, u, r = _tc_prep1(x, lin1_W, lin1_b, Wa, g_att_r)

    gu, re = _sc_gather_gate(u, r, src, dst)
    ee = _tc_ee_gate(gu, edge_attr, re, WbT, g_att_l)
    s2 = _sc_segsum(ee, dst)
    acc = _sc_rows(x1, src, dst, ee, s2)
    x2, xp1, as1, ad1 = _tc_post_gate(
        acc, x1, g_lin2_W, g_bias, gru0_Wih, gru0_bih, gru0_Whh, gru0_bhh,
        atom_W[0], atom_att_src[0], atom_att_dst[0])

    ee1, s2b = _sc_p1_atom(as1, ad1, src, dst)
    acc1 = _sc_rows(xp1, src, dst, ee1, s2b)
    x3, xp2, as2, ad2 = _tc_post_atom(
        acc1, x2, atom_b[0], agru_Wih[0], agru_bih[0], agru_Whh[0],
        agru_bhh[0], atom_W[1], atom_att_src[1], atom_att_dst[1])

    ee2, s2c = _sc_p1_atom(as2, ad2, src, dst)
    acc2 = _sc_rows(xp2, src, dst, ee2, s2c)
    x4, xp_s, a_s, _ = _tc_post_atom(
        acc2, x3, atom_b[1], agru_Wih[1], agru_bih[1], agru_Whh[1],
        agru_bhh[1], mol_W, mol_att_src, mol_att_src)

    batch_p = jnp.concatenate([batch.astype(jnp.int32),
                               jnp.full((NP - N,), B, jnp.int32)])
    x4 = jnp.pad(x4, ((0, NP - N), (0, 0)))
    xp_s = jnp.pad(xp_s, ((0, NP - N), (0, 0)))
    a_s = jnp.pad(a_s, (0, NP - N))
    out0 = _tc_mol0(x4, batch_p)
    ov = _tc_mol(xp_s, a_s, batch_p, out0, descriptors, mol_W, mol_att_dst,
                 mol_b, mgru_Wih, mgru_bih, mgru_Whh, mgru_bhh,
                 m1_W, m1_b, bn1_g, bn1_b, m2_W, m2_b, bn2_g, bn2_b,
                 m3_W, m3_b, bn3_g, bn3_b, c1_W, c1_b, c2_W, c2_b, c3_W)
    return ov[:, None] + c3_b[None, :]


# speedup vs baseline: 19.4591x; 2.0190x over previous
import functools
import jax
import jax.numpy as jnp
from jax import lax
from jax.experimental import pallas as pl
from jax.experimental.pallas import tpu as pltpu
from jax.experimental.pallas import tpu_sc as plsc

N = 10000
NP = 10240
E = 320000
H = 128
B = 512
ED = 16

NC = 2
NS = 16
EC = E // NC
ET = EC // NS
C = 128
NCH = ET // C
TAIL = ET - NCH * C
SLICE = NP // NS

_MESH = functools.partial(
    plsc.VectorSubcoreMesh, core_axis_name="c", subcore_axis_name="s")
_SC_PARAMS = pltpu.CompilerParams(needs_layout_passes=False)


def _dot_t(a, w):
    return lax.dot_general(a, w, (((1,), (1,)), ((), ())),
                           preferred_element_type=jnp.float32)


def _leaky(v):
    return jnp.where(v > 0, v, 0.01 * v)


def _elu(v):
    return jnp.where(v > 0, v, jnp.exp(v) - 1.0)


def _sig(v):
    return 1.0 / (1.0 + jnp.exp(-v))


def _gru(inp, hid, Wih, bih, Whh, bhh):
    gi = _dot_t(inp, Wih) + bih
    gh = _dot_t(hid, Whh) + bhh
    r = _sig(gi[:, :H] + gh[:, :H])
    z = _sig(gi[:, H:2 * H] + gh[:, H:2 * H])
    n_ = jnp.tanh(gi[:, 2 * H:] + r * gh[:, 2 * H:])
    return (1.0 - z) * n_ + z * hid


def _bn(v, g, b):
    m = v.mean(0)
    var = v.var(0)
    return (v - m) / jnp.sqrt(var + 1e-5) * g + b


def _prep1_body(x_ref, w_ref, b_ref, wa_ref, ar_ref, x1_ref, u_ref, r_ref):
    x1 = _leaky(_dot_t(x_ref[...], w_ref[...]) + b_ref[...][None, :])
    x1_ref[...] = x1
    u_ref[...] = _dot_t(x1, wa_ref[...])
    r_ref[...] = jnp.dot(x1, ar_ref[...])


def _tc_prep1(x, lin1_W, lin1_b, Wa, att_r):
    return pl.pallas_call(
        _prep1_body,
        out_shape=(
            jax.ShapeDtypeStruct((N, H), jnp.float32),
            jax.ShapeDtypeStruct((N, H), jnp.float32),
            jax.ShapeDtypeStruct((N,), jnp.float32),
        ),
    )(x, lin1_W, lin1_b, Wa, att_r)


_BE = 512


def _ee_gate_body(gu_ref, ea_ref, re_ref, wbt_ref, al_ref, ee_ref):
    v = jnp.dot(ea_ref[...], wbt_ref[...], preferred_element_type=jnp.float32)
    t = _leaky(gu_ref[...] + v)
    ell = _leaky(jnp.dot(t, al_ref[...]) + re_ref[...])
    ee_ref[...] = jnp.exp(ell)


def _tc_ee_gate(gu, ea, re, WbT, att_l):
    grid = E // _BE
    return pl.pallas_call(
        _ee_gate_body,
        grid=(grid,),
        in_specs=[
            pl.BlockSpec((_BE, H), lambda i: (i, 0)),
            pl.BlockSpec((_BE, ED), lambda i: (i, 0)),
            pl.BlockSpec((_BE,), lambda i: (i,)),
            pl.BlockSpec((ED, H), lambda i: (0, 0)),
            pl.BlockSpec((H,), lambda i: (0,)),
        ],
        out_specs=pl.BlockSpec((_BE,), lambda i: (i,)),
        out_shape=jax.ShapeDtypeStruct((E,), jnp.float32),
    )(gu, ea, re, WbT, att_l)


def _post_core(accsum, h, xprev, wih, bih, whh, bhh, nw, nas, nad,
               x_ref, xp_ref, as_ref, ad_ref):
    xnew = jnp.maximum(_gru(h, xprev, wih, bih, whh, bhh), 0.0)
    x_ref[...] = xnew
    xp = _dot_t(xnew, nw)
    xp_ref[...] = xp
    as_ref[...] = jnp.dot(xp, nas)
    ad_ref[...] = jnp.dot(xp, nad)


def _post_gate_body(acc_ref, s_ref, x_ref, w2_ref, gb_ref, wih_ref, bih_ref,
                    whh_ref, bhh_ref, nw_ref, nas_ref, nad_ref,
                    xo_ref, xp_ref, as_ref, ad_ref):
    den = s_ref[0, :N] + s_ref[1, :N] + 1e-16
    accsum = (acc_ref[0, :N, :] + acc_ref[1, :N, :]) / den[:, None]
    h = _elu(_dot_t(accsum, w2_ref[...]) + gb_ref[...][None, :])
    _post_core(accsum, h, x_ref[...], wih_ref[...], bih_ref[...],
               whh_ref[...], bhh_ref[...], nw_ref[...], nas_ref[...],
               nad_ref[...], xo_ref, xp_ref, as_ref, ad_ref)


def _post_atom_body(acc_ref, s_ref, x_ref, ab_ref, wih_ref, bih_ref,
                    whh_ref, bhh_ref, nw_ref, nas_ref, nad_ref,
                    xo_ref, xp_ref, as_ref, ad_ref):
    den = s_ref[0, :N] + s_ref[1, :N] + 1e-16
    accsum = (acc_ref[0, :N, :] + acc_ref[1, :N, :]) / den[:, None]
    h = _elu(accsum + ab_ref[...][None, :])
    _post_core(accsum, h, x_ref[...], wih_ref[...], bih_ref[...],
               whh_ref[...], bhh_ref[...], nw_ref[...], nas_ref[...],
               nad_ref[...], xo_ref, xp_ref, as_ref, ad_ref)


_POST_OUT = (
    jax.ShapeDtypeStruct((N, H), jnp.float32),
    jax.ShapeDtypeStruct((N, H), jnp.float32),
    jax.ShapeDtypeStruct((N,), jnp.float32),
    jax.ShapeDtypeStruct((N,), jnp.float32),
)


def _tc_post_gate(acc, s2, x1, w2, gb, wih, bih, whh, bhh, nw, nas, nad):
    return pl.pallas_call(_post_gate_body, out_shape=_POST_OUT)(
        acc, s2, x1, w2, gb, wih, bih, whh, bhh, nw, nas, nad)


def _tc_post_atom(acc, s2, xprev, ab, wih, bih, whh, bhh, nw, nas, nad):
    return pl.pallas_call(_post_atom_body, out_shape=_POST_OUT)(
        acc, s2, xprev, ab, wih, bih, whh, bhh, nw, nas, nad)


_CW = 1280


def _build_z(batch_ref, z_ref):
    for k in range(NP // _CW):
        sl = pl.ds(k * _CW, _CW)
        bi = batch_ref[sl]
        io2 = lax.broadcasted_iota(jnp.int32, (B, _CW), 0)
        z_ref[:, sl] = (bi[None, :] == io2).astype(jnp.float32)


def _mol0_body(x_ref, batch_ref, o_ref, z_ref):
    _build_z(batch_ref, z_ref)
    o_ref[...] = jnp.maximum(
        jnp.dot(z_ref[...], x_ref[...], preferred_element_type=jnp.float32),
        0.0)


def _tc_mol0(x4p, batch_p):
    return pl.pallas_call(
        _mol0_body,
        out_shape=jax.ShapeDtypeStruct((B, H), jnp.float32),
        scratch_shapes=[pltpu.VMEM((B, NP), jnp.float32)],
    )(x4p, batch_p)


def _mol_body(xps_ref, as_ref, batch_ref, out0_ref, desc_ref,
              molW_ref, mad_ref, mb_ref,
              mwih_ref, mbih_ref, mwhh_ref, mbhh_ref,
              m1w_ref, m1b_ref, g1_ref, b1_ref,
              m2w_ref, m2b_ref, g2_ref, b2_ref,
              m3w_ref, m3b_ref, g3_ref, b3_ref,
              c1w_ref, c1b_ref, c2w_ref, c2b_ref, c3w_ref,
              o_ref, z_ref):
    xps = xps_ref[...]
    a_s = as_ref[...]
    _build_z(batch_ref, z_ref)
    Z = z_ref[...]

    out = out0_ref[...]
    for _t in range(2):
        a_d = jnp.dot(_dot_t(out, molW_ref[...]), mad_ref[...])
        ell = _leaky(a_s + jnp.dot(a_d, Z))
        ee = jnp.exp(ell)
        s_b = jnp.dot(Z, ee)
        alpha = ee / (jnp.dot(s_b, Z) + 1e-16)
        h = jnp.dot(Z, xps * alpha[:, None],
                    preferred_element_type=jnp.float32) + mb_ref[...][None, :]
        h = _elu(h)
        out = jnp.maximum(
            _gru(h, out, mwih_ref[...], mbih_ref[...], mwhh_ref[...],
                 mbhh_ref[...]), 0.0)

    d = jnp.maximum(_bn(_dot_t(desc_ref[...], m1w_ref[...]) + m1b_ref[...],
                        g1_ref[...], b1_ref[...]), 0.0)
    d = jnp.maximum(_bn(_dot_t(d, m2w_ref[...]) + m2b_ref[...],
                        g2_ref[...], b2_ref[...]), 0.0)
    d = jnp.maximum(_bn(_dot_t(d, m3w_ref[...]) + m3b_ref[...],
                        g3_ref[...], b3_ref[...]), 0.0)
    comb = jnp.concatenate([d, out], axis=1)
    o = jnp.maximum(_dot_t(comb, c1w_ref[...]) + c1b_ref[...], 0.0)
    o = jnp.maximum(_dot_t(o, c2w_ref[...]) + c2b_ref[...], 0.0)
    o_ref[...] = jnp.dot(o, c3w_ref[0, :])


def _tc_mol(xps, a_s, batch, out0, desc, molW, mad, mb,
            mwih, mbih, mwhh, mbhh, m1w, m1b, g1, b1, m2w, m2b, g2, b2,
            m3w, m3b, g3, b3, c1w, c1b, c2w, c2b, c3w):
    return pl.pallas_call(
        _mol_body,
        out_shape=jax.ShapeDtypeStruct((B,), jnp.float32),
        scratch_shapes=[pltpu.VMEM((B, NP), jnp.float32)],
    )(xps, a_s, batch, out0, desc, molW, mad, mb, mwih, mbih, mwhh, mbhh,
      m1w, m1b, g1, b1, m2w, m2b, g2, b2, m3w, m3b, g3, b3,
      c1w, c1b, c2w, c2b, c3w)


def _sc_gather_body(u_h, r_h, src_h, dst_h, gu_h, re_h,
                    r_vm, src_vm, dst_vm, re_vm, rows0, rows1, sem0, sem1):
    c = lax.axis_index("c")
    s = lax.axis_index("s")
    tbase = c * EC + s * ET
    pltpu.sync_copy(r_h, r_vm.at[pl.ds(0, N)])
    pltpu.sync_copy(src_h.at[pl.ds(tbase, ET)], src_vm)
    pltpu.sync_copy(dst_h.at[pl.ds(tbase, ET)], dst_vm)

    def pre(j, carry):
        def grp(g, carry2):
            gl = pl.ds(j * 128 + g * 16, 16)
            re_vm[gl] = plsc.load_gather(r_vm, [dst_vm[gl]])
            return carry2
        lax.fori_loop(0, 8, grp, 0)
        return carry
    lax.fori_loop(0, ET // 128, pre, 0)
    pltpu.sync_copy(re_vm, re_h.at[pl.ds(tbase, ET)])

    rows = (rows0, rows1)
    sems = (sem0, sem1)

    def gidx(j):
        return u_h.at[src_vm.at[pl.ds(j * C, C)]]

    pltpu.async_copy(gidx(0), rows0, sem0)

    def pair(j2, carry):
        j = 2 * j2
        pltpu.async_copy(gidx(j + 1), rows1, sem1)
        pltpu.make_async_copy(gidx(j), rows0, sem0).wait()
        pltpu.sync_copy(rows0, gu_h.at[pl.ds(tbase + j * C, C), :])

        @pl.when(j2 < NCH // 2 - 1)
        def _():
            pltpu.async_copy(gidx(j + 2), rows0, sem0)
        pltpu.make_async_copy(gidx(j + 1), rows1, sem1).wait()
        pltpu.sync_copy(rows1, gu_h.at[pl.ds(tbase + (j + 1) * C, C), :])
        return carry
    lax.fori_loop(0, NCH // 2, pair, 0)

    tb = NCH * C
    pltpu.async_copy(u_h.at[src_vm.at[pl.ds(tb, TAIL)]],
                     rows0.at[pl.ds(0, TAIL), :], sem0).wait()
    pltpu.sync_copy(rows0.at[pl.ds(0, TAIL), :],
                    gu_h.at[pl.ds(tbase + tb, TAIL), :])


def _sc_gather_gate(u, r, src, dst):
    f = pl.kernel(
        _sc_gather_body,
        out_type=(
            jax.ShapeDtypeStruct((E, H), jnp.float32),
            jax.ShapeDtypeStruct((E,), jnp.float32),
        ),
        mesh=_MESH(),
        compiler_params=_SC_PARAMS,
        scratch_types=[
            pltpu.VMEM((NP,), jnp.float32),
            pltpu.VMEM((ET,), jnp.int32),
            pltpu.VMEM((ET,), jnp.int32),
            pltpu.VMEM((ET,), jnp.float32),
            pltpu.VMEM((C, H), jnp.float32),
            pltpu.VMEM((C, H), jnp.float32),
            pltpu.SemaphoreType.DMA,
            pltpu.SemaphoreType.DMA,
        ],
    )
    return f(u, r, src, dst)


def _zero_vec(ref, nwords):
    z = jnp.zeros((16,), jnp.float32)

    def body(i, carry):
        ref[pl.ds(i * 16, 16)] = z
        return carry
    lax.fori_loop(0, nwords // 16, body, 0)


def _seg_add_loop(ee_vm, dst2d, dt16, s_spm):
    def addj(j, carry):
        pltpu.sync_copy(ee_vm.at[pl.ds(j * C, C)],
                        s_spm.at[dst2d.at[j]], add=True)
        return carry
    lax.fori_loop(0, NCH, addj, 0)
    pltpu.sync_copy(ee_vm.at[pl.ds(NCH * C, TAIL)],
                    s_spm.at[dt16], add=True)


def _sc_p1_atom_body(as_h, ad_h, src_h, dst_h, ee_h, s_h,
                     av_vm, bv_vm, src_vm, dst_vm, ee_vm, dst2d, dt16, zb,
                     s_spm):
    c = lax.axis_index("c")
    s = lax.axis_index("s")
    tbase = c * EC + s * ET
    pltpu.sync_copy(as_h, av_vm.at[pl.ds(0, N)])
    pltpu.sync_copy(ad_h, bv_vm.at[pl.ds(0, N)])
    pltpu.sync_copy(src_h.at[pl.ds(tbase, ET)], src_vm)
    pltpu.sync_copy(dst_h.at[pl.ds(tbase, ET)], dst_vm)
    _zero_vec(zb, SLICE)
    pltpu.sync_copy(zb, s_spm.at[pl.ds(s * SLICE, SLICE)])
    plsc.subcore_barrier()

    def pre(j, carry):
        for g in range(8):
            sl = pl.ds(g * 16, 16)
            gl = pl.ds(j * C + g * 16, 16)
            di = dst_vm[gl]
            dst2d[j, sl] = di
            a = (plsc.load_gather(av_vm, [src_vm[gl]])
                 + plsc.load_gather(bv_vm, [di]))
            ee_vm[gl] = jnp.exp(jnp.where(a > 0, a, 0.01 * a))
        return carry
    lax.fori_loop(0, NCH, pre, 0)
    gl = pl.ds(NCH * C, TAIL)
    di = dst_vm[gl]
    dt16[pl.ds(0, 16)] = di
    a = (plsc.load_gather(av_vm, [src_vm[gl]])
         + plsc.load_gather(bv_vm, [di]))
    ee_vm[gl] = jnp.exp(jnp.where(a > 0, a, 0.01 * a))

    pltpu.sync_copy(ee_vm, ee_h.at[pl.ds(tbase, ET)])
    _seg_add_loop(ee_vm, dst2d, dt16, s_spm)
    plsc.subcore_barrier()
    pltpu.sync_copy(s_spm.at[pl.ds(s * SLICE, SLICE)], zb)
    pltpu.sync_copy(zb, s_h.at[c, pl.ds(s * SLICE, SLICE)])


def _sc_p1_atom(asrc, adst, src, dst):
    f = pl.kernel(
        _sc_p1_atom_body,
        out_type=(
            jax.ShapeDtypeStruct((E,), jnp.float32),
            jax.ShapeDtypeStruct((NC, NP), jnp.float32),
        ),
        mesh=_MESH(),
        compiler_params=_SC_PARAMS,
        scratch_types=[
            pltpu.VMEM((NP,), jnp.float32),
            pltpu.VMEM((NP,), jnp.float32),
            pltpu.VMEM((ET,), jnp.int32),
            pltpu.VMEM((ET,), jnp.int32),
            pltpu.VMEM((ET,), jnp.float32),
            pltpu.VMEM((NCH, C), jnp.int32),
            pltpu.VMEM((16,), jnp.int32),
            pltpu.VMEM((SLICE,), jnp.float32),
            pltpu.VMEM_SHARED((NP,), jnp.float32),
        ],
    )
    return f(asrc, adst, src, dst)


def _sc_seg_body(ee_h, dst_h, s_h, dst_vm, ee_vm, dst2d, dt16, zb, s_spm):
    c = lax.axis_index("c")
    s = lax.axis_index("s")
    tbase = c * EC + s * ET
    pltpu.sync_copy(dst_h.at[pl.ds(tbase, ET)], dst_vm)
    pltpu.sync_copy(ee_h.at[pl.ds(tbase, ET)], ee_vm)
    _zero_vec(zb, SLICE)
    pltpu.sync_copy(zb, s_spm.at[pl.ds(s * SLICE, SLICE)])
    plsc.subcore_barrier()

    def pre(j, carry):
        for g in range(8):
            dst2d[j, pl.ds(g * 16, 16)] = dst_vm[pl.ds(j * C + g * 16, 16)]
        return carry
    lax.fori_loop(0, NCH, pre, 0)
    dt16[pl.ds(0, 16)] = dst_vm[pl.ds(NCH * C, TAIL)]

    _seg_add_loop(ee_vm, dst2d, dt16, s_spm)
    plsc.subcore_barrier()
    pltpu.sync_copy(s_spm.at[pl.ds(s * SLICE, SLICE)], zb)
    pltpu.sync_copy(zb, s_h.at[c, pl.ds(s * SLICE, SLICE)])


def _sc_segsum(ee, dst):
    f = pl.kernel(
        _sc_seg_body,
        out_type=jax.ShapeDtypeStruct((NC, NP), jnp.float32),
        mesh=_MESH(),
        compiler_params=_SC_PARAMS,
        scratch_types=[
            pltpu.VMEM((ET,), jnp.int32),
            pltpu.VMEM((ET,), jnp.float32),
            pltpu.VMEM((NCH, C), jnp.int32),
            pltpu.VMEM((16,), jnp.int32),
            pltpu.VMEM((SLICE,), jnp.float32),
            pltpu.VMEM_SHARED((NP,), jnp.float32),
        ],
    )
    return f(ee, dst)


def _sc_rows_body(tab_h, src_h, dst_h, ee_h, acc_h,
                  srcb0, srcb1, dstb0, dstb1, eeb0, eeb1, rows0, rows1,
                  isem0, isem1, gsem0, gsem1, acc_spm):
    c = lax.axis_index("c")
    s = lax.axis_index("s")
    tbase = c * EC + s * ET
    srcb = (srcb0, srcb1)
    dstb = (dstb0, dstb1)
    eeb = (eeb0, eeb1)
    rows = (rows0, rows1)
    isem = (isem0, isem1)
    gsem = (gsem0, gsem1)

    def idx_issue(j, b):
        base = tbase + j * C
        pltpu.async_copy(src_h.at[pl.ds(base, C)], srcb[b], isem[b])
        pltpu.async_copy(dst_h.at[pl.ds(base, C)], dstb[b], isem[b])
        pltpu.async_copy(ee_h.at[pl.ds(base, C)], eeb[b], isem[b])

    def idx_wait(j, b):
        base = tbase + j * C
        pltpu.make_async_copy(src_h.at[pl.ds(base, C)], srcb[b],
                              isem[b]).wait()
        pltpu.make_async_copy(dst_h.at[pl.ds(base, C)], dstb[b],
                              isem[b]).wait()
        pltpu.make_async_copy(ee_h.at[pl.ds(base, C)], eeb[b],
                              isem[b]).wait()

    z = jnp.zeros((16,), jnp.float32)

    def zrow_i(i, carry):
        for k in range(H // 16):
            rows0[i, pl.ds(k * 16, 16)] = z
        return carry
    lax.fori_loop(0, C, zrow_i, 0)

    def zcp(b_, carry):
        pltpu.sync_copy(rows0, acc_spm.at[pl.ds(s * SLICE + b_ * C, C), :])
        return carry
    lax.fori_loop(0, SLICE // C, zcp, 0)
    plsc.subcore_barrier()

    def scale(rref, eref):
        def grp2(g, carry):
            av = eref[pl.ds(g * 16, 16)]
            for lane in range(16):
                a = av[lane]
                i = g * 16 + lane
                for k in range(H // 16):
                    slk = pl.ds(k * 16, 16)
                    rref[i, slk] = rref[i, slk] * a
            return carry
        lax.fori_loop(0, C // 16, grp2, 0)

    idx_issue(0, 0)
    idx_wait(0, 0)
    pltpu.async_copy(tab_h.at[srcb0], rows0, gsem0)
    idx_issue(1, 1)

    def pair(j2, carry):
        j = 2 * j2
        idx_wait(j + 1, 1)
        pltpu.async_copy(tab_h.at[srcb1], rows1, gsem1)
        pltpu.make_async_copy(tab_h.at[srcb0], rows0, gsem0).wait()
        scale(rows0, eeb0)
        pltpu.sync_copy(rows0, acc_spm.at[dstb0], add=True)

        @pl.when(j2 < NCH // 2 - 1)
        def _():
            idx_issue(j + 2, 0)
            idx_wait(j + 2, 0)
            pltpu.async_copy(tab_h.at[srcb0], rows0, gsem0)
        pltpu.make_async_copy(tab_h.at[srcb1], rows1, gsem1).wait()
        scale(rows1, eeb1)
        pltpu.sync_copy(rows1, acc_spm.at[dstb1], add=True)

        @pl.when(j2 < NCH // 2 - 1)
        def _():
            idx_issue(j + 3, 1)
        return carry
    lax.fori_loop(0, NCH // 2, pair, 0)

    base = tbase + NCH * C
    pltpu.sync_copy(src_h.at[pl.ds(base, TAIL)], srcb0.at[pl.ds(0, TAIL)])
    pltpu.sync_copy(dst_h.at[pl.ds(base, TAIL)], dstb1.at[pl.ds(0, TAIL)])
    pltpu.sync_copy(ee_h.at[pl.ds(base, TAIL)], eeb0.at[pl.ds(0, TAIL)])
    pltpu.async_copy(tab_h.at[srcb0.at[pl.ds(0, TAIL)]],
                     rows0.at[pl.ds(0, TAIL), :], gsem0).wait()
    av = eeb0[pl.ds(0, TAIL)]
    dt16 = dstb1[pl.ds(0, TAIL)]
    dstb0[pl.ds(0, 16)] = dt16
    for lane in range(TAIL):
        a = av[lane]
        for k in range(H // 16):
            slk = pl.ds(k * 16, 16)
            rows0[lane, slk] = rows0[lane, slk] * a
    pltpu.sync_copy(rows0.at[pl.ds(0, TAIL), :],
                    acc_spm.at[dstb0.at[pl.ds(0, TAIL)]], add=True)
    plsc.subcore_barrier()

    def out_cp(b_, carry):
        off = s * SLICE + b_ * C
        pltpu.sync_copy(acc_spm.at[pl.ds(off, C), :], rows0)
        pltpu.sync_copy(rows0, acc_h.at[c, pl.ds(off, C), :])
        return carry
    lax.fori_loop(0, SLICE // C, out_cp, 0)


def _sc_rows(table, src, dst, ee):
    f = pl.kernel(
        _sc_rows_body,
        out_type=jax.ShapeDtypeStruct((NC, NP, H), jnp.float32),
        mesh=_MESH(),
        compiler_params=_SC_PARAMS,
        scratch_types=[
            pltpu.VMEM((C,), jnp.int32),
            pltpu.VMEM((C,), jnp.int32),
            pltpu.VMEM((C,), jnp.int32),
            pltpu.VMEM((C,), jnp.int32),
            pltpu.VMEM((C,), jnp.float32),
            pltpu.VMEM((C,), jnp.float32),
            pltpu.VMEM((C, H), jnp.float32),
            pltpu.VMEM((C, H), jnp.float32),
            pltpu.SemaphoreType.DMA,
            pltpu.SemaphoreType.DMA,
            pltpu.SemaphoreType.DMA,
            pltpu.SemaphoreType.DMA,
            pltpu.VMEM_SHARED((NP, H), jnp.float32),
        ],
    )
    return f(table, src, dst, ee)


def kernel(x, edge_index, edge_attr, descriptors, batch,
           lin1_W, lin1_b, g_lin1_W, g_lin2_W, g_att_l, g_att_r, g_bias,
           gru0_Wih, gru0_bih, gru0_Whh, gru0_bhh,
           atom_W, atom_att_src, atom_att_dst, atom_b,
           agru_Wih, agru_bih, agru_Whh, agru_bhh,
           mol_W, mol_att_src, mol_att_dst, mol_b,
           mgru_Wih, mgru_bih, mgru_Whh, mgru_bhh,
           m1_W, m1_b, bn1_g, bn1_b,
           m2_W, m2_b, bn2_g, bn2_b,
           m3_W, m3_b, bn3_g, bn3_b,
           c1_W, c1_b, c2_W, c2_b, c3_W, c3_b):
    src = edge_index[0]
    dst = edge_index[1]
    Wa = g_lin1_W[:, :H]
    WbT = g_lin1_W[:, H:].T

    x1, u, r = _tc_prep1(x, lin1_W, lin1_b, Wa, g_att_r)

    gu, re = _sc_gather_gate(u, r, src, dst)
    ee = _tc_ee_gate(gu, edge_attr, re, WbT, g_att_l)
    s2 = _sc_segsum(ee, dst)
    acc = _sc_rows(x1, src, dst, ee)
    x2, xp1, as1, ad1 = _tc_post_gate(
        acc, s2, x1, g_lin2_W, g_bias, gru0_Wih, gru0_bih, gru0_Whh,
        gru0_bhh, atom_W[0], atom_att_src[0], atom_att_dst[0])

    ee1, s2b = _sc_p1_atom(as1, ad1, src, dst)
    acc1 = _sc_rows(xp1, src, dst, ee1)
    x3, xp2, as2, ad2 = _tc_post_atom(
        acc1, s2b, x2, atom_b[0], agru_Wih[0], agru_bih[0], agru_Whh[0],
        agru_bhh[0], atom_W[1], atom_att_src[1], atom_att_dst[1])

    ee2, s2c = _sc_p1_atom(as2, ad2, src, dst)
    acc2 = _sc_rows(xp2, src, dst, ee2)
    x4, xp_s, a_s, _ = _tc_post_atom(
        acc2, s2c, x3, atom_b[1], agru_Wih[1], agru_bih[1], agru_Whh[1],
        agru_bhh[1], mol_W, mol_att_src, mol_att_src)

    batch_p = jnp.concatenate([batch.astype(jnp.int32),
                               jnp.full((NP - N,), B, jnp.int32)])
    x4 = jnp.pad(x4, ((0, NP - N), (0, 0)))
    xp_s = jnp.pad(xp_s, ((0, NP - N), (0, 0)))
    a_s = jnp.pad(a_s, (0, NP - N))
    out0 = _tc_mol0(x4, batch_p)
    ov = _tc_mol(xp_s, a_s, batch_p, out0, descriptors, mol_W, mol_att_dst,
                 mol_b, mgru_Wih, mgru_bih, mgru_Whh, mgru_bhh,
                 m1_W, m1_b, bn1_g, bn1_b, m2_W, m2_b, bn2_g, bn2_b,
                 m3_W, m3_b, bn3_g, bn3_b, c1_W, c1_b, c2_W, c2_b, c3_W)
    return ov[:, None] + c3_b[None, :]
